# Initial kernel scaffold; baseline (speedup 1.0000x reference)
import jax
import jax.numpy as jnp
from jax.experimental import pallas as pl


def kernel(x, edge_index, batch, Wl1, bl1, Wr1, Wl2, bl2, Wr2):
    raise NotImplementedError("write your pallas kernel here")


# speedup vs baseline: 10.5338x; 10.5338x over previous
import functools

import jax
import jax.numpy as jnp
from jax import lax
from jax.experimental import pallas as pl
from jax.experimental.pallas import tpu as pltpu
from jax.experimental.pallas import tpu_sc as plsc

N_NODES = 10000
N_EDGES = 320000
D_IN = 128
D_HID = 256
N_CLASSES = 64

NC = 2
NS = 16
N_PAD = 10240
BLK = 128
SUB = 8
EDGE_ROWS = 2500
EDGE_ROWS_PAD = 2560
BLKS_PER_TILE = EDGE_ROWS_PAD // (NC * NS)
IDX_OUTER = BLKS_PER_TILE // SUB
NODES_PER_TILE = N_PAD // NS


def _seg_sum_body(d, with_count, x_hbm, src_hbm, dst_hbm, *refs):
    if with_count:
        out_hbm, cnt_hbm, src_idx, dst_idx, rows, ones, zcnt, acc, cnt_acc, sem = refs
    else:
        out_hbm, src_idx, dst_idx, rows, acc, sem = refs
        cnt_hbm = cnt_acc = ones = zcnt = None

    cid = lax.axis_index("c")
    sid = lax.axis_index("s")
    wid = cid * NS + sid

    def zero_rows(i, _):
        for k in range(d // 16):
            rows[i, pl.ds(k * 16, 16)] = jnp.zeros((16,), jnp.float32)
        return _
    lax.fori_loop(0, BLK, zero_rows, None)
    if with_count:
        def fill_cnt(i, _):
            ones[pl.ds(i * 16, 16)] = jnp.full((16,), 1.0, jnp.float32)
            zcnt[pl.ds(i * 16, 16)] = jnp.zeros((16,), jnp.float32)
            return _
        lax.fori_loop(0, BLK // 16, fill_cnt, None)
        def zero_zcnt(i, _):
            zcnt[pl.ds(i * 16, 16)] = jnp.zeros((16,), jnp.float32)
            return _
        lax.fori_loop(0, NODES_PER_TILE // 16, zero_zcnt, None)

    base_n = sid * NODES_PER_TILE
    for k in range(NODES_PER_TILE // BLK):
        pltpu.sync_copy(rows, acc.at[pl.ds(base_n + k * BLK, BLK)])
    if with_count:
        pltpu.sync_copy(zcnt, cnt_acc.at[pl.ds(base_n, NODES_PER_TILE)])
    plsc.subcore_barrier()

    base_e = wid * IDX_OUTER
    pltpu.sync_copy(src_hbm.at[pl.ds(base_e, IDX_OUTER)], src_idx)
    pltpu.sync_copy(dst_hbm.at[pl.ds(base_e, IDX_OUTER)], dst_idx)

    nblk = jnp.where(wid == NC * NS - 1,
                     BLKS_PER_TILE - (EDGE_ROWS_PAD - EDGE_ROWS),
                     BLKS_PER_TILE)

    def edge_block(j, _):
        r = j // SUB
        s = j % SUB
        pltpu.async_copy(x_hbm.at[src_idx.at[r, s]], rows, sem).wait()
        pltpu.sync_copy(rows, acc.at[dst_idx.at[r, s]], add=True)
        if with_count:
            pltpu.sync_copy(ones, cnt_acc.at[dst_idx.at[r, s]], add=True)
        return _
    lax.fori_loop(0, nblk, edge_block, None)
    plsc.subcore_barrier()

    pltpu.sync_copy(acc.at[pl.ds(base_n, NODES_PER_TILE)],
                    out_hbm.at[cid, pl.ds(base_n, NODES_PER_TILE)])
    if with_count:
        pltpu.sync_copy(cnt_acc.at[pl.ds(base_n, NODES_PER_TILE)],
                        cnt_hbm.at[cid, pl.ds(base_n, NODES_PER_TILE)])


def _make_seg_sum(d, with_count):
    mesh = plsc.VectorSubcoreMesh(core_axis_name="c", subcore_axis_name="s",
                                  num_cores=NC, num_subcores=NS)
    out_type = [jax.ShapeDtypeStruct((NC, N_PAD, d), jnp.float32)]
    scratch = [
        pltpu.VMEM((IDX_OUTER, SUB, BLK), jnp.int32),
        pltpu.VMEM((IDX_OUTER, SUB, BLK), jnp.int32),
        pltpu.VMEM((BLK, d), jnp.float32),
    ]
    if with_count:
        out_type.append(jax.ShapeDtypeStruct((NC, N_PAD), jnp.float32))
        scratch += [
            pltpu.VMEM((BLK,), jnp.float32),
            pltpu.VMEM((NODES_PER_TILE,), jnp.float32),
        ]
    scratch += [pltpu.VMEM_SHARED((N_PAD, d), jnp.float32)]
    if with_count:
        scratch += [pltpu.VMEM_SHARED((N_PAD,), jnp.float32)]
    scratch += [pltpu.SemaphoreType.DMA]
    return pl.kernel(functools.partial(_seg_sum_body, d, with_count),
                     out_type=out_type, mesh=mesh, scratch_types=scratch,
                     name=f"sage_seg_sum_d{d}")


def _layer1_tc(P_ref, cnt_ref, x_ref, Wl1_ref, bl1_ref, Wr1_ref,
               Wl2_ref, bl2_ref, Wr2_ref, p_ref, q_ref):
    c = (cnt_ref[0] + cnt_ref[1]).reshape(-1, 1)
    mean = (P_ref[0] + P_ref[1]) * (1.0 / jnp.maximum(c, 1.0))
    h = jnp.dot(mean, Wl1_ref[...], preferred_element_type=jnp.float32)
    h = h + jnp.dot(x_ref[...], Wr1_ref[...], preferred_element_type=jnp.float32)
    h = jnp.maximum(h + bl1_ref[...], 0.0)
    p = jnp.dot(h, Wl2_ref[...], preferred_element_type=jnp.float32)
    p_ref[...] = jnp.concatenate(
        [p, jnp.zeros_like(p)], axis=1)
    q_ref[...] = (jnp.dot(h, Wr2_ref[...], preferred_element_type=jnp.float32)
                  + bl2_ref[...])


def _layer2_tc(P2_ref, cnt_ref, q_ref, o_ref):
    c = (cnt_ref[0] + cnt_ref[1]).reshape(-1, 1)
    agg = (P2_ref[0] + P2_ref[1])[:, :N_CLASSES]
    z = agg * (1.0 / jnp.maximum(c, 1.0)) + q_ref[...]
    m = jnp.max(z, axis=1, keepdims=True)
    e = jnp.exp(z - m)
    s = jnp.sum(e, axis=1, keepdims=True)
    o_ref[...] = z - m - jnp.log(s)


_ROWS_B = 1024


def kernel(x, edge_index, batch, Wl1, bl1, Wr1, Wl2, bl2, Wr2):
    del batch
    src = edge_index[0].astype(jnp.int32)
    dst = edge_index[1].astype(jnp.int32)
    pad = EDGE_ROWS_PAD * BLK - N_EDGES
    src3d = jnp.pad(src, (0, pad)).reshape(EDGE_ROWS_PAD // SUB, SUB, BLK)
    dst3d = jnp.pad(dst, (0, pad)).reshape(EDGE_ROWS_PAD // SUB, SUB, BLK)
    xp = jnp.pad(x, ((0, N_PAD - N_NODES), (0, 0)))

    P1, cnt = _make_seg_sum(D_IN, True)(xp, src3d, dst3d)

    grid = (N_PAD // _ROWS_B,)
    p, q = pl.pallas_call(
        _layer1_tc,
        grid=grid,
        in_specs=[
            pl.BlockSpec((NC, _ROWS_B, D_IN), lambda i: (0, i, 0)),
            pl.BlockSpec((NC, _ROWS_B), lambda i: (0, i)),
            pl.BlockSpec((_ROWS_B, D_IN), lambda i: (i, 0)),
            pl.BlockSpec((D_IN, D_HID), lambda i: (0, 0)),
            pl.BlockSpec((1, D_HID), lambda i: (0, 0)),
            pl.BlockSpec((D_IN, D_HID), lambda i: (0, 0)),
            pl.BlockSpec((D_HID, N_CLASSES), lambda i: (0, 0)),
            pl.BlockSpec((1, N_CLASSES), lambda i: (0, 0)),
            pl.BlockSpec((D_HID, N_CLASSES), lambda i: (0, 0)),
        ],
        out_specs=[
            pl.BlockSpec((_ROWS_B, 2 * N_CLASSES), lambda i: (i, 0)),
            pl.BlockSpec((_ROWS_B, N_CLASSES), lambda i: (i, 0)),
        ],
        out_shape=[
            jax.ShapeDtypeStruct((N_PAD, 2 * N_CLASSES), jnp.float32),
            jax.ShapeDtypeStruct((N_PAD, N_CLASSES), jnp.float32),
        ],
    )(P1, cnt, xp, Wl1, bl1.reshape(1, D_HID), Wr1,
      Wl2, bl2.reshape(1, N_CLASSES), Wr2)

    (P2,) = _make_seg_sum(2 * N_CLASSES, False)(p, src3d, dst3d)

    out = pl.pallas_call(
        _layer2_tc,
        grid=grid,
        in_specs=[
            pl.BlockSpec((NC, _ROWS_B, 2 * N_CLASSES), lambda i: (0, i, 0)),
            pl.BlockSpec((NC, _ROWS_B), lambda i: (0, i)),
            pl.BlockSpec((_ROWS_B, N_CLASSES), lambda i: (i, 0)),
        ],
        out_specs=pl.BlockSpec((_ROWS_B, N_CLASSES), lambda i: (i, 0)),
        out_shape=jax.ShapeDtypeStruct((N_PAD, N_CLASSES), jnp.float32),
    )(P2, cnt, q)
    return out[:N_NODES]
